# baseline (speedup 1.0000x reference)
import functools

import jax
import jax.numpy as jnp
from jax import lax
from jax.experimental import pallas as pl
from jax.experimental.pallas import tpu as pltpu
from jax.experimental.pallas import tpu_sc as plsc

VOCAB = 1000000
N_EMBED = 64
BLOCK = 2048
B, T = 4, 2048

_INFO = plsc.get_sparse_core_info()
_NW = _INFO.num_cores * _INFO.num_subcores
_ROWS = B * T
_PER_W = _ROWS // _NW
_CHUNK = 128
_NCH = _PER_W // _CHUNK
_VECS = N_EMBED // _INFO.num_lanes


def _sc_embed(table, idx2, pos):
    mesh = plsc.VectorSubcoreMesh(core_axis_name="c", subcore_axis_name="s")

    @functools.partial(
        pl.kernel,
        mesh=mesh,
        out_type=jax.ShapeDtypeStruct((_ROWS, N_EMBED), jnp.float32),
        compiler_params=pltpu.CompilerParams(use_tc_tiling_on_sc=False),
        scratch_types=[
            pltpu.VMEM((_NCH, _CHUNK), jnp.int32),
            pltpu.VMEM((_CHUNK, N_EMBED), jnp.float32),
            pltpu.VMEM((_CHUNK, N_EMBED), jnp.float32),
            pltpu.VMEM((_PER_W, N_EMBED), jnp.float32),
            pltpu.SemaphoreType.DMA,
            pltpu.SemaphoreType.DMA,
            pltpu.SemaphoreType.DMA,
        ],
    )
    def k(table_hbm, idx_hbm, pos_hbm, out_hbm,
          idx_v, rows0, rows1, pos_v, sem0, sem1, sem_out):
        wid = lax.axis_index("s") * _INFO.num_cores + lax.axis_index("c")
        row0 = wid * _PER_W
        t0 = (wid % (T // _PER_W)) * _PER_W

        pltpu.sync_copy(idx_hbm.at[pl.ds(wid * _NCH, _NCH)], idx_v)
        cp0 = pltpu.async_copy(table_hbm.at[idx_v.at[0]], rows0, sem0)
        cp1 = pltpu.async_copy(table_hbm.at[idx_v.at[1]], rows1, sem1)
        pltpu.sync_copy(pos_hbm.at[pl.ds(t0, _PER_W)], pos_v)

        def add_chunk(rows, off):
            def body(r, _):
                for c in range(_VECS):
                    sl = pl.ds(c * _INFO.num_lanes, _INFO.num_lanes)
                    rows[r, sl] = rows[r, sl] + pos_v[off + r, sl]
                return 0
            lax.fori_loop(0, _CHUNK, body, 0, unroll=2)

        cp0.wait()
        add_chunk(rows0, 0)
        out0 = pltpu.async_copy(rows0, out_hbm.at[pl.ds(row0, _CHUNK)], sem_out)
        cp1.wait()
        add_chunk(rows1, _CHUNK)
        pltpu.sync_copy(rows1, out_hbm.at[pl.ds(row0 + _CHUNK, _CHUNK)])
        out0.wait()

    return k(table, idx2, pos)


def kernel(idx, token_table, pos_table):
    idx2 = idx.astype(jnp.int32).reshape(_NW * _NCH, _CHUNK)
    out = _sc_embed(token_table, idx2, pos_table)
    return out.reshape(B, T, N_EMBED)


# speedup vs baseline: 1.6736x; 1.6736x over previous
import functools

import jax
import jax.numpy as jnp
from jax import lax
from jax.experimental import pallas as pl
from jax.experimental.pallas import tpu as pltpu
from jax.experimental.pallas import tpu_sc as plsc

VOCAB = 1000000
N_EMBED = 64
BLOCK = 2048
B, T = 4, 2048

_INFO = plsc.get_sparse_core_info()
_NW = _INFO.num_cores * _INFO.num_subcores
_ROWS = B * T
_PER_W = _ROWS // _NW
_WAVE = 16
_NWAVES = _PER_W // _WAVE
_VECS = N_EMBED // _INFO.num_lanes


def _sc_embed(table, idx_flat, pos):
    mesh = plsc.VectorSubcoreMesh(core_axis_name="c", subcore_axis_name="s")

    @functools.partial(
        pl.kernel,
        mesh=mesh,
        out_type=jax.ShapeDtypeStruct((_ROWS, N_EMBED), jnp.float32),
        compiler_params=pltpu.CompilerParams(use_tc_tiling_on_sc=True),
        scratch_types=[
            pltpu.VMEM((_PER_W,), jnp.int32),
            pltpu.VMEM((_PER_W, N_EMBED), jnp.float32),
            pltpu.VMEM((_PER_W, N_EMBED), jnp.float32),
            pltpu.SemaphoreType.DMA,
        ],
    )
    def k(table_hbm, idx_hbm, pos_hbm, out_hbm, idx_v, rows_v, pos_v, sem):
        wid = lax.axis_index("s") * _INFO.num_cores + lax.axis_index("c")
        row0 = wid * _PER_W
        t0 = (wid % (T // _PER_W)) * _PER_W

        pltpu.sync_copy(idx_hbm.at[pl.ds(row0, _PER_W)], idx_v)
        pltpu.sync_copy(pos_hbm.at[pl.ds(t0, _PER_W)], pos_v)

        def drain(nrows):
            pltpu.make_async_copy(
                table_hbm.at[pl.ds(0, nrows)], rows_v.at[pl.ds(0, nrows)], sem
            ).wait()

        def wave(g, _):
            ivec = idx_v[pl.ds(g * _WAVE, _WAVE)]
            for j in range(_WAVE):
                r = g * _WAVE + j
                pltpu.async_copy(table_hbm.at[ivec[j]], rows_v.at[r], sem)

            @pl.when(g > 0)
            def _():
                drain(_WAVE)
            return 0

        lax.fori_loop(0, _NWAVES, wave, 0)
        drain(_WAVE)

        def add(r, _):
            for c in range(_VECS):
                sl = pl.ds(c * _INFO.num_lanes, _INFO.num_lanes)
                rows_v[r, sl] = rows_v[r, sl] + pos_v[r, sl]
            return 0
        lax.fori_loop(0, _PER_W, add, 0, unroll=2)

        pltpu.sync_copy(rows_v, out_hbm.at[pl.ds(row0, _PER_W)])

    return k(table, idx_flat, pos)


def kernel(idx, token_table, pos_table):
    idx_flat = idx.astype(jnp.int32).reshape(_ROWS)
    out = _sc_embed(token_table, idx_flat, pos_table)
    return out.reshape(B, T, N_EMBED)


# speedup vs baseline: 3.6227x; 2.1646x over previous
import functools

import jax
import jax.numpy as jnp
from jax import lax
from jax.experimental import pallas as pl
from jax.experimental.pallas import tpu as pltpu
from jax.experimental.pallas import tpu_sc as plsc

VOCAB = 1000000
N_EMBED = 64
BLOCK = 2048
B, T = 4, 2048

_INFO = plsc.get_sparse_core_info()
_L = _INFO.num_lanes
_NW = _INFO.num_cores * _INFO.num_subcores
_ROWS = B * T
_PER_W = _ROWS // _NW
_VECS = N_EMBED // _L
_TCOL = 128


def _sc_embed(table_t, idx_flat, pos):
    mesh = plsc.VectorSubcoreMesh(core_axis_name="c", subcore_axis_name="s")

    @functools.partial(
        pl.kernel,
        mesh=mesh,
        out_type=jax.ShapeDtypeStruct((_ROWS, N_EMBED), jnp.float32),
        compiler_params=pltpu.CompilerParams(
            use_tc_tiling_on_sc=True, needs_layout_passes=False),
        scratch_types=[
            pltpu.VMEM((_PER_W,), jnp.int32),
            pltpu.VMEM((2, N_EMBED, _TCOL), jnp.float32),
            pltpu.VMEM((_PER_W, N_EMBED), jnp.float32),
            pltpu.VMEM((_PER_W, N_EMBED), jnp.float32),
            pltpu.SemaphoreType.DMA,
        ],
    )
    def k(table_hbm, idx_hbm, pos_hbm, out_hbm,
          idx_v, chunk_v, rows_v, pos_v, sem):
        wid = lax.axis_index("s") * _INFO.num_cores + lax.axis_index("c")
        row0 = wid * _PER_W
        t0 = (wid % (T // _PER_W)) * _PER_W

        pltpu.sync_copy(idx_hbm.at[pl.ds(row0, _PER_W)], idx_v)
        pltpu.sync_copy(pos_hbm.at[pl.ds(t0, _PER_W)], pos_v)

        lane = lax.iota(jnp.int32, _L)
        nwaves = _PER_W // _L

        def fire(ti, parity):
            q0 = (ti // _TCOL) * _TCOL
            pltpu.async_copy(
                table_hbm.at[:, pl.ds(pl.multiple_of(q0, _TCOL), _TCOL)],
                chunk_v.at[parity], sem)

        def drain():
            pltpu.make_async_copy(
                table_hbm.at[:, pl.ds(0, _TCOL)], chunk_v.at[0], sem).wait()

        fire(idx_v[pl.ds(0, _L)][0], 0)

        def body(g, _):
            ivec = idx_v[pl.ds(g * _L, _L)]
            for j in range(_L):
                if j < _L - 1:
                    fire(ivec[j + 1], (j + 1) % 2)
                else:
                    @pl.when(g + 1 < nwaves)
                    def _():
                        nvec = idx_v[pl.ds((g + 1) * _L, _L)]
                        fire(nvec[0], 0)
                drain()
                r = g * _L + j
                m = lax.rem(ivec[j], _TCOL)
                buf = chunk_v.at[j % 2]
                for v in range(_VECS):
                    cvec = plsc.load_gather(buf, [v * _L + lane, m + 0 * lane])
                    sl = pl.ds(v * _L, _L)
                    rows_v[r, sl] = cvec + pos_v[r, sl]
            return 0

        lax.fori_loop(0, nwaves, body, 0)

        pltpu.sync_copy(rows_v, out_hbm.at[pl.ds(row0, _PER_W)])

    return k(table_t, idx_flat, pos)


def kernel(idx, token_table, pos_table):
    idx_flat = idx.astype(jnp.int32).reshape(_ROWS)
    out = _sc_embed(token_table.T, idx_flat, pos_table)
    return out.reshape(B, T, N_EMBED)


# speedup vs baseline: 5.1448x; 1.4201x over previous
import functools

import jax
import jax.numpy as jnp
from jax import lax
from jax.experimental import pallas as pl
from jax.experimental.pallas import tpu as pltpu
from jax.experimental.pallas import tpu_sc as plsc

VOCAB = 1000000
N_EMBED = 64
BLOCK = 2048
B, T = 4, 2048

_INFO = plsc.get_sparse_core_info()
_L = _INFO.num_lanes
_NW = _INFO.num_cores * _INFO.num_subcores
_ROWS = B * T
_PER_W = _ROWS // _NW
_VECS = N_EMBED // _L
_TCOL = 128
_DEPTH = 4
_AHEAD = _DEPTH - 1


def _sc_embed(table_t, idx_flat, pos):
    mesh = plsc.VectorSubcoreMesh(core_axis_name="c", subcore_axis_name="s")

    @functools.partial(
        pl.kernel,
        mesh=mesh,
        out_type=jax.ShapeDtypeStruct((_ROWS, N_EMBED), jnp.float32),
        compiler_params=pltpu.CompilerParams(
            use_tc_tiling_on_sc=True, needs_layout_passes=False),
        scratch_types=[
            pltpu.VMEM((_PER_W,), jnp.int32),
            pltpu.VMEM((_DEPTH, N_EMBED, _TCOL), jnp.float32),
            pltpu.VMEM((_PER_W, N_EMBED), jnp.float32),
            pltpu.VMEM((_PER_W, N_EMBED), jnp.float32),
            pltpu.SemaphoreType.DMA,
        ],
    )
    def k(table_hbm, idx_hbm, pos_hbm, out_hbm,
          idx_v, chunk_v, rows_v, pos_v, sem):
        wid = lax.axis_index("s") * _INFO.num_cores + lax.axis_index("c")
        row0 = wid * _PER_W
        t0 = (wid % (T // _PER_W)) * _PER_W

        pltpu.sync_copy(idx_hbm.at[pl.ds(row0, _PER_W)], idx_v)
        pltpu.sync_copy(pos_hbm.at[pl.ds(t0, _PER_W)], pos_v)

        lane = lax.iota(jnp.int32, _L)
        nwaves = _PER_W // _L

        def fire(ti, slot):
            q0 = (ti // _TCOL) * _TCOL
            pltpu.async_copy(
                table_hbm.at[:, pl.ds(pl.multiple_of(q0, _TCOL), _TCOL)],
                chunk_v.at[slot], sem)

        def drain():
            pltpu.make_async_copy(
                table_hbm.at[:, pl.ds(0, _TCOL)], chunk_v.at[0], sem).wait()

        ivec0 = idx_v[pl.ds(0, _L)]
        for j in range(_AHEAD):
            fire(ivec0[j], j)

        def body(g, _):
            ivec = idx_v[pl.ds(g * _L, _L)]
            nvec = idx_v[pl.ds(lax.min(g + 1, nwaves - 1) * _L, _L)]
            for j in range(_L):
                jn = j + _AHEAD
                if jn < _L:
                    fire(ivec[jn], jn % _DEPTH)
                else:
                    @pl.when(g + 1 < nwaves)
                    def _():
                        fire(nvec[jn - _L], jn % _DEPTH)
                drain()
                r = g * _L + j
                m = lax.rem(ivec[j], _TCOL)
                buf = chunk_v.at[j % _DEPTH]
                for v in range(_VECS):
                    cvec = plsc.load_gather(buf, [v * _L + lane, m + 0 * lane])
                    sl = pl.ds(v * _L, _L)
                    rows_v[r, sl] = cvec + pos_v[r, sl]
            return 0

        lax.fori_loop(0, nwaves, body, 0)

        pltpu.sync_copy(rows_v, out_hbm.at[pl.ds(row0, _PER_W)])

    return k(table_t, idx_flat, pos)


def kernel(idx, token_table, pos_table):
    idx_flat = idx.astype(jnp.int32).reshape(_ROWS)
    out = _sc_embed(token_table.T, idx_flat, pos_table)
    return out.reshape(B, T, N_EMBED)


# speedup vs baseline: 5.3873x; 1.0471x over previous
import functools

import jax
import jax.numpy as jnp
from jax import lax
from jax.experimental import pallas as pl
from jax.experimental.pallas import tpu as pltpu
from jax.experimental.pallas import tpu_sc as plsc

VOCAB = 1000000
N_EMBED = 64
BLOCK = 2048
B, T = 4, 2048

_INFO = plsc.get_sparse_core_info()
_L = _INFO.num_lanes
_NW = _INFO.num_cores * _INFO.num_subcores
_ROWS = B * T
_PER_W = _ROWS // _NW
_VECS = N_EMBED // _L
_TCOL = 128
_DEPTH = 6
_AHEAD = _DEPTH - 1


def _sc_embed(table_t, idx_flat, pos):
    mesh = plsc.VectorSubcoreMesh(core_axis_name="c", subcore_axis_name="s")

    @functools.partial(
        pl.kernel,
        mesh=mesh,
        out_type=jax.ShapeDtypeStruct((_ROWS, N_EMBED), jnp.float32),
        compiler_params=pltpu.CompilerParams(
            use_tc_tiling_on_sc=True, needs_layout_passes=False),
        scratch_types=[
            pltpu.VMEM((_PER_W,), jnp.int32),
            pltpu.VMEM((_DEPTH, N_EMBED, _TCOL), jnp.float32),
            pltpu.VMEM((_PER_W, N_EMBED), jnp.float32),
            pltpu.VMEM((_PER_W, N_EMBED), jnp.float32),
            pltpu.SemaphoreType.DMA,
        ],
    )
    def k(table_hbm, idx_hbm, pos_hbm, out_hbm,
          idx_v, chunk_v, rows_v, pos_v, sem):
        wid = lax.axis_index("s") * _INFO.num_cores + lax.axis_index("c")
        row0 = wid * _PER_W
        t0 = (wid % (T // _PER_W)) * _PER_W

        pltpu.sync_copy(idx_hbm.at[pl.ds(row0, _PER_W)], idx_v)
        pltpu.sync_copy(pos_hbm.at[pl.ds(t0, _PER_W)], pos_v)

        lane = lax.iota(jnp.int32, _L)
        nwaves = _PER_W // _L

        def fire(ti, slot):
            q0 = (ti // _TCOL) * _TCOL
            pltpu.async_copy(
                table_hbm.at[:, pl.ds(pl.multiple_of(q0, _TCOL), _TCOL)],
                chunk_v.at[slot], sem)

        def drain():
            pltpu.make_async_copy(
                table_hbm.at[:, pl.ds(0, _TCOL)], chunk_v.at[0], sem).wait()

        ivec0 = idx_v[pl.ds(0, _L)]
        for j in range(_AHEAD):
            fire(ivec0[j], j)

        def body(g, _):
            ivec = idx_v[pl.ds(g * _L, _L)]
            nvec = idx_v[pl.ds(lax.min(g + 1, nwaves - 1) * _L, _L)]
            for j in range(_L):
                jn = j + _AHEAD
                if jn < _L:
                    fire(ivec[jn], lax.rem(g * _L + jn, _DEPTH))
                else:
                    @pl.when(g + 1 < nwaves)
                    def _():
                        fire(nvec[jn - _L], lax.rem(g * _L + jn, _DEPTH))
                drain()
                r = g * _L + j
                m = lax.rem(ivec[j], _TCOL)
                buf = chunk_v.at[lax.rem(r, _DEPTH)]
                for v in range(_VECS):
                    cvec = plsc.load_gather(buf, [v * _L + lane, m + 0 * lane])
                    sl = pl.ds(v * _L, _L)
                    rows_v[r, sl] = cvec + pos_v[r, sl]
            return 0

        lax.fori_loop(0, nwaves, body, 0)

        pltpu.sync_copy(rows_v, out_hbm.at[pl.ds(row0, _PER_W)])

    return k(table_t, idx_flat, pos)


def kernel(idx, token_table, pos_table):
    idx_flat = idx.astype(jnp.int32).reshape(_ROWS)
    out = _sc_embed(token_table.T, idx_flat, pos_table)
    return out.reshape(B, T, N_EMBED)
